# baseline (speedup 1.0000x reference)
import functools

import jax
import jax.numpy as jnp
from jax.experimental import pallas as pl
from jax.experimental.pallas import tpu as pltpu

_BS = 5
_PAD = _BS - 1


def _shift_down(a, d):
    return jnp.concatenate(
        [jnp.zeros_like(a[..., :d, :]), a[..., :-d, :]], axis=-2)


def _shift_right(a, d):
    return jnp.concatenate(
        [jnp.zeros_like(a[..., :, :d]), a[..., :, :-d]], axis=-1)


def _block_mask(gamma, valid, CB, H, W):
    pltpu.prng_seed(pl.program_id(0))
    bits = pltpu.bitcast(pltpu.prng_random_bits((CB, H, W)), jnp.uint32)
    thr = (jnp.minimum(gamma, 1.0) * 4294967040.0).astype(jnp.uint32)
    m = jnp.where(bits < thr, valid[None], 0.0)
    s1 = jnp.maximum(m, _shift_down(m, 1))
    s2 = jnp.maximum(s1, _shift_down(s1, 2))
    rm = jnp.maximum(s2, _shift_down(m, 4))
    t1 = jnp.maximum(rm, _shift_right(rm, 1))
    t2 = jnp.maximum(t1, _shift_right(t1, 2))
    p = jnp.maximum(t2, _shift_right(rm, 4))
    return 1.0 - p


def _apply_body(gamma_ref, valid_ref, x_ref, o_ref, count_ref, *, CB, H, W):
    bm = _block_mask(gamma_ref[0, 0], valid_ref[0, 0], CB, H, W)
    o_ref[...] = x_ref[...] * bm[None]

    @pl.when(pl.program_id(0) == 0)
    def _init():
        count_ref[0, 0] = 0.0

    count_ref[0, 0] += jnp.sum(bm)


def _rescale_body(scale_ref, y_ref, o_ref):
    o_ref[...] = y_ref[...] * scale_ref[0, 0]


def kernel(x, gamma):
    B, C, H, W = x.shape
    hh, ww = H - _PAD, W - _PAD
    CB = C
    grid = (B,)
    g = jnp.asarray(gamma, jnp.float32).reshape(1, 1)
    countM = float(B * C * H * W)

    iota_h = jax.lax.broadcasted_iota(jnp.int32, (H, W), 0)
    iota_w = jax.lax.broadcasted_iota(jnp.int32, (H, W), 1)
    valid = ((iota_h < hh) & (iota_w < ww)).astype(jnp.float32)
    valid = valid.reshape(1, 1, H, W)

    smem_scalar = pl.BlockSpec((1, 1), lambda i: (0, 0),
                               memory_space=pltpu.SMEM)
    valid_spec = pl.BlockSpec((1, 1, H, W), lambda i: (0, 0, 0, 0))
    big_spec = pl.BlockSpec((1, CB, H, W), lambda i: (i, 0, 0, 0))

    out_raw, count_ones = pl.pallas_call(
        functools.partial(_apply_body, CB=CB, H=H, W=W),
        grid=grid,
        in_specs=[smem_scalar, valid_spec, big_spec],
        out_specs=[big_spec, smem_scalar],
        out_shape=[
            jax.ShapeDtypeStruct((B, C, H, W), jnp.float32),
            jax.ShapeDtypeStruct((1, 1), jnp.float32),
        ],
    )(g, valid, x)

    def _done(args):
        out, _ = args
        return out

    def _rescale(args):
        out, count = args
        scale = (countM / count).reshape(1, 1)
        return pl.pallas_call(
            _rescale_body,
            grid=grid,
            in_specs=[smem_scalar, big_spec],
            out_specs=big_spec,
            out_shape=jax.ShapeDtypeStruct((B, C, H, W), jnp.float32),
        )(scale, out)

    return jax.lax.cond(count_ones[0, 0] == countM, _done, _rescale,
                        (out_raw, count_ones))
